# baseline (speedup 1.0000x reference)
import functools

import jax
import jax.numpy as jnp
from jax import lax
from jax.experimental import pallas as pl
from jax.experimental.pallas import tpu as pltpu
from jax.experimental.pallas import tpu_sc as plsc

MAX_SPIKE_TIME = 100000.0
B = 512
IN_SIZE = 512
OUT_SIZE = 128
L = 16
NCHUNK = IN_SIZE // L
OCHUNK = OUT_SIZE // L
NW = 32
ROWS_PER_W = B // NW
CSZ = 128
NBIG = IN_SIZE // CSZ
KC_PER_BIG = CSZ // L
HG = NCHUNK // 2

_GATHER_DNUMS = lax.GatherDimensionNumbers(
    offset_dims=(), collapsed_slice_dims=(0,), start_index_map=(0,))


def _lane_bcast(v, lane):
    idx = jnp.full((L,), lane, jnp.int32)
    return lax.gather(v, idx[:, None], dimension_numbers=_GATHER_DNUMS,
                      slice_sizes=(1,),
                      mode=lax.GatherScatterMode.PROMISE_IN_BOUNDS)


def _snn_body(x_hbm, w_hbm, out_hbm, w_v, x_v, sx_v, sxn_v, sidx_v, orow_v):
    wid = lax.axis_index("s") * 2 + lax.axis_index("c")
    pltpu.sync_copy(w_hbm, w_v)
    iota = lax.iota(jnp.int32, L)

    def row_body(r, _):
        row = wid * ROWS_PER_W + r
        pltpu.sync_copy(x_hbm.at[row], x_v)

        neg1 = jnp.full((L,), -1.0, jnp.float32)
        for c in range(NCHUNK):
            sx_v[pl.ds(c * L, L)] = neg1

        sxn_v[pl.ds(IN_SIZE - L, L)] = jnp.full((L,), MAX_SPIKE_TIME,
                                                jnp.float32)

        for half in range(2):
            xi_c = [x_v[pl.ds((half * HG + g) * L, L)] for g in range(HG)]

            def jchunk_body(jc, ranks, _xi=xi_c):
                xj = x_v[pl.ds(jc * L, L)]
                ranks = list(ranks)
                for l in range(L):
                    xjl = _lane_bcast(xj, l)
                    for g in range(HG):
                        ranks[g] = ranks[g] + (xjl < _xi[g]).astype(jnp.int32)
                return tuple(ranks)

            init = (jnp.zeros((L,), jnp.int32),) * HG
            ranks = lax.fori_loop(0, NCHUNK, jchunk_body, init)
            for g in range(HG):
                ic = half * HG + g
                rank = ranks[g]
                xi = xi_c[g]
                plsc.store_scatter(sx_v, [rank], xi)
                plsc.store_scatter(sidx_v, [rank], iota + ic * L)
                plsc.store_scatter(sxn_v, [rank - 1], xi, mask=rank >= 1)

        hole = jnp.zeros((L,), jnp.int32)
        for c in range(NCHUNK):
            hole = jnp.maximum(
                hole, (sx_v[pl.ds(c * L, L)] == -1.0).astype(jnp.int32))
        has_tie = jnp.max(hole) > 0

        @pl.when(has_tie)
        def tie_fallback():
            sxn_v[pl.ds(IN_SIZE - L, L)] = jnp.full((L,), MAX_SPIKE_TIME,
                                                    jnp.float32)

            def ichunk_body(ic, _):
                xi = x_v[pl.ds(ic * L, L)]
                ii = iota + ic * L

                def j_le(jc, rank):
                    xj = x_v[pl.ds(jc * L, L)]
                    for l in range(L):
                        xjl = _lane_bcast(xj, l)
                        rank = rank + (xjl <= xi).astype(jnp.int32)
                    return rank

                def j_lt(jc, rank):
                    xj = x_v[pl.ds(jc * L, L)]
                    for l in range(L):
                        xjl = _lane_bcast(xj, l)
                        rank = rank + (xjl < xi).astype(jnp.int32)
                    return rank

                rank = lax.fori_loop(0, ic, j_le, jnp.zeros((L,), jnp.int32))
                rank = lax.fori_loop(ic + 1, NCHUNK, j_lt, rank)
                for l in range(L):
                    xjl = _lane_bcast(xi, l)
                    cond = (xjl < xi) | ((xjl == xi) & (iota > l))
                    rank = rank + cond.astype(jnp.int32)
                plsc.store_scatter(sx_v, [rank], xi)
                plsc.store_scatter(sidx_v, [rank], ii)
                plsc.store_scatter(sxn_v, [rank - 1], xi, mask=rank >= 1)
                return 0

            lax.fori_loop(0, NCHUNK, ichunk_body, 0)

        mn = [jnp.full((L,), MAX_SPIKE_TIME, jnp.float32)] * OCHUNK
        carryw = [jnp.zeros((L,), jnp.float32)] * OCHUNK
        carrywi = [jnp.zeros((L,), jnp.float32)] * OCHUNK
        for c in range(NBIG):
            cw_l = carryw
            cwi_l = carrywi

            def kchunk_body(kc, carry, _cw=cw_l, _cwi=cwi_l, _c=c):
                cw = list(carry[0:OCHUNK])
                cwi = list(carry[OCHUNK:2 * OCHUNK])
                mnl = list(carry[2 * OCHUNK:3 * OCHUNK])
                base = _c * CSZ + kc * L
                adr_c = sidx_v[pl.ds(base, L)]
                sx_c = sx_v[pl.ds(base, L)]
                sxn_c = sxn_v[pl.ds(base, L)]
                for l in range(L):
                    ridx = _lane_bcast(adr_c, l)
                    xk = _lane_bcast(sx_c, l)
                    xn = _lane_bcast(sxn_c, l)
                    for o in range(OCHUNK):
                        wv = plsc.load_gather(w_v, [ridx, iota + o * L])
                        cw[o] = cw[o] + wv
                        cwi[o] = cwi[o] + wv * xk
                        if _c == 0:
                            wsum = cw[o]
                            wisum = cwi[o]
                        else:
                            wsum = cw[o] + _cw[o]
                            wisum = cwi[o] + _cwi[o]
                        den = jnp.maximum(wsum - 1.0, 1e-10)
                        t = wisum / den
                        t = jnp.where(wsum < 1.0, MAX_SPIKE_TIME, t)
                        t = jnp.where(t < xk, MAX_SPIKE_TIME, t)
                        t = jnp.where(t > xn, MAX_SPIKE_TIME, t)
                        mnl[o] = jnp.minimum(mnl[o], t)
                return tuple(cw) + tuple(cwi) + tuple(mnl)

            zero = jnp.zeros((L,), jnp.float32)
            init = (zero,) * (2 * OCHUNK) + tuple(mn)
            res = lax.fori_loop(0, KC_PER_BIG, kchunk_body, init)
            mn = list(res[2 * OCHUNK:3 * OCHUNK])
            if c == 0:
                carryw = [res[o] for o in range(OCHUNK)]
                carrywi = [res[OCHUNK + o] for o in range(OCHUNK)]
            elif c < NBIG - 1:
                carryw = [carryw[o] + res[o] for o in range(OCHUNK)]
                carrywi = [carrywi[o] + res[OCHUNK + o] for o in range(OCHUNK)]

        for o in range(OCHUNK):
            orow_v[pl.ds(o * L, L)] = mn[o]
        pltpu.sync_copy(orow_v, out_hbm.at[row])
        return 0

    lax.fori_loop(0, ROWS_PER_W, row_body, 0)


@jax.jit
def kernel(layer_in, weight):
    mesh = plsc.VectorSubcoreMesh(core_axis_name="c", subcore_axis_name="s")
    f = pl.kernel(
        _snn_body,
        out_type=jax.ShapeDtypeStruct((B, OUT_SIZE), jnp.float32),
        mesh=mesh,
        scratch_types=[
            pltpu.VMEM((IN_SIZE, OUT_SIZE), jnp.float32),
            pltpu.VMEM((IN_SIZE,), jnp.float32),
            pltpu.VMEM((IN_SIZE,), jnp.float32),
            pltpu.VMEM((IN_SIZE,), jnp.float32),
            pltpu.VMEM((IN_SIZE,), jnp.int32),
            pltpu.VMEM((OUT_SIZE,), jnp.float32),
        ],
        compiler_params=pltpu.CompilerParams(needs_layout_passes=False),
    )
    return f(layer_in, weight)


# speedup vs baseline: 4.5841x; 4.5841x over previous
import jax
import jax.numpy as jnp
from jax import lax
from jax.experimental import pallas as pl
from jax.experimental.pallas import tpu as pltpu
from jax.experimental.pallas import tpu_sc as plsc

MAX_SPIKE_TIME = 100000.0
B = 512
IN_SIZE = 512
OUT_SIZE = 128
L = 16
NCHUNK = IN_SIZE // L
OCHUNK = OUT_SIZE // L
NW = 32
ROWS_PER_W = B // NW
CSZ = 128
NBIG = IN_SIZE // CSZ
KC_PER_BIG = CSZ // L
RB = 8

_GATHER_DNUMS = lax.GatherDimensionNumbers(
    offset_dims=(), collapsed_slice_dims=(0,), start_index_map=(0,))


def _lane_bcast(v, lane):
    idx = jnp.full((L,), lane, jnp.int32)
    return lax.gather(v, idx[:, None], dimension_numbers=_GATHER_DNUMS,
                      slice_sizes=(1,),
                      mode=lax.GatherScatterMode.PROMISE_IN_BOUNDS)


def _rank_body(x_ref, xt_ref, o_ref):
    jlt = (lax.broadcasted_iota(jnp.int32, (IN_SIZE, IN_SIZE), 0)
           < lax.broadcasted_iota(jnp.int32, (IN_SIZE, IN_SIZE), 1))
    for r in range(RB):
        xi = x_ref[pl.ds(r, 1), :]
        xj = xt_ref[0, :, pl.ds(r, 1)]
        lt = xj < xi
        eq = xj == xi
        cnt = jnp.where(lt | (eq & jlt), 1.0, 0.0)
        rank = jnp.sum(cnt, axis=0)
        o_ref[pl.ds(r, 1), :] = rank[None, :].astype(jnp.int32)


def _ranks_tc(x, xt3):
    return pl.pallas_call(
        _rank_body,
        grid=(B // RB,),
        in_specs=[
            pl.BlockSpec((RB, IN_SIZE), lambda g: (g, 0)),
            pl.BlockSpec((1, IN_SIZE, RB), lambda g: (g, 0, 0)),
        ],
        out_specs=pl.BlockSpec((RB, IN_SIZE), lambda g: (g, 0)),
        out_shape=jax.ShapeDtypeStruct((B, IN_SIZE), jnp.int32),
    )(x, xt3)


def _snn_body(x_hbm, w_hbm, rk_hbm, out_hbm,
              w_v, x_v, rk_v, sx_v, sxn_v, sidx_v, orow_v):
    wid = lax.axis_index("s") * 2 + lax.axis_index("c")
    pltpu.sync_copy(w_hbm, w_v)
    iota = lax.iota(jnp.int32, L)

    def row_body(r, _):
        row = wid * ROWS_PER_W + r
        pltpu.sync_copy(x_hbm.at[row], x_v)
        pltpu.sync_copy(rk_hbm.at[row], rk_v)

        sxn_v[pl.ds(IN_SIZE - L, L)] = jnp.full((L,), MAX_SPIKE_TIME,
                                                jnp.float32)

        def chunk_body(c, _):
            rank = rk_v[pl.ds(c * L, L)]
            xi = x_v[pl.ds(c * L, L)]
            plsc.store_scatter(sx_v, [rank], xi)
            plsc.store_scatter(sidx_v, [rank], iota + c * L)
            plsc.store_scatter(sxn_v, [rank - 1], xi, mask=rank >= 1)
            return 0

        lax.fori_loop(0, NCHUNK, chunk_body, 0)

        mn = [jnp.full((L,), MAX_SPIKE_TIME, jnp.float32)] * OCHUNK
        carryw = [jnp.zeros((L,), jnp.float32)] * OCHUNK
        carrywi = [jnp.zeros((L,), jnp.float32)] * OCHUNK
        for c in range(NBIG):
            cw_l = carryw
            cwi_l = carrywi

            def kchunk_body(kc, carry, _cw=cw_l, _cwi=cwi_l, _c=c):
                cw = list(carry[0:OCHUNK])
                cwi = list(carry[OCHUNK:2 * OCHUNK])
                mnl = list(carry[2 * OCHUNK:3 * OCHUNK])
                base = _c * CSZ + kc * L
                adr_c = sidx_v[pl.ds(base, L)]
                sx_c = sx_v[pl.ds(base, L)]
                sxn_c = sxn_v[pl.ds(base, L)]
                for l in range(L):
                    ridx = _lane_bcast(adr_c, l)
                    xk = _lane_bcast(sx_c, l)
                    xn = _lane_bcast(sxn_c, l)
                    for o in range(OCHUNK):
                        wv = plsc.load_gather(w_v, [ridx, iota + o * L])
                        cw[o] = cw[o] + wv
                        cwi[o] = cwi[o] + wv * xk
                        if _c == 0:
                            wsum = cw[o]
                            wisum = cwi[o]
                        else:
                            wsum = cw[o] + _cw[o]
                            wisum = cwi[o] + _cwi[o]
                        den = jnp.maximum(wsum - 1.0, 1e-10)
                        t = wisum / den
                        t = jnp.where(wsum < 1.0, MAX_SPIKE_TIME, t)
                        t = jnp.where(t < xk, MAX_SPIKE_TIME, t)
                        t = jnp.where(t > xn, MAX_SPIKE_TIME, t)
                        mnl[o] = jnp.minimum(mnl[o], t)
                return tuple(cw) + tuple(cwi) + tuple(mnl)

            zero = jnp.zeros((L,), jnp.float32)
            init = (zero,) * (2 * OCHUNK) + tuple(mn)
            res = lax.fori_loop(0, KC_PER_BIG, kchunk_body, init)
            mn = list(res[2 * OCHUNK:3 * OCHUNK])
            if c == 0:
                carryw = [res[o] for o in range(OCHUNK)]
                carrywi = [res[OCHUNK + o] for o in range(OCHUNK)]
            elif c < NBIG - 1:
                carryw = [carryw[o] + res[o] for o in range(OCHUNK)]
                carrywi = [carrywi[o] + res[OCHUNK + o] for o in range(OCHUNK)]

        for o in range(OCHUNK):
            orow_v[pl.ds(o * L, L)] = mn[o]
        pltpu.sync_copy(orow_v, out_hbm.at[row])
        return 0

    lax.fori_loop(0, ROWS_PER_W, row_body, 0)


@jax.jit
def kernel(layer_in, weight):
    xt3 = layer_in.T.reshape(IN_SIZE, B // RB, RB).transpose(1, 0, 2)
    ranks = _ranks_tc(layer_in, xt3)
    mesh = plsc.VectorSubcoreMesh(core_axis_name="c", subcore_axis_name="s")
    f = pl.kernel(
        _snn_body,
        out_type=jax.ShapeDtypeStruct((B, OUT_SIZE), jnp.float32),
        mesh=mesh,
        scratch_types=[
            pltpu.VMEM((IN_SIZE, OUT_SIZE), jnp.float32),
            pltpu.VMEM((IN_SIZE,), jnp.float32),
            pltpu.VMEM((IN_SIZE,), jnp.int32),
            pltpu.VMEM((IN_SIZE,), jnp.float32),
            pltpu.VMEM((IN_SIZE,), jnp.float32),
            pltpu.VMEM((IN_SIZE,), jnp.int32),
            pltpu.VMEM((OUT_SIZE,), jnp.float32),
        ],
        compiler_params=pltpu.CompilerParams(needs_layout_passes=False),
    )
    return f(layer_in, weight, ranks)


# speedup vs baseline: 4.7627x; 1.0390x over previous
import jax
import jax.numpy as jnp
from jax import lax
from jax.experimental import pallas as pl
from jax.experimental.pallas import tpu as pltpu
from jax.experimental.pallas import tpu_sc as plsc

MAX_SPIKE_TIME = 100000.0
B = 512
IN_SIZE = 512
OUT_SIZE = 128
L = 16
NCHUNK = IN_SIZE // L
OCHUNK = OUT_SIZE // L
NW = 32
ROWS_PER_W = B // NW
CSZ = 128
NBIG = IN_SIZE // CSZ
KC_PER_BIG = CSZ // L
RB = 8

_GATHER_DNUMS = lax.GatherDimensionNumbers(
    offset_dims=(), collapsed_slice_dims=(0,), start_index_map=(0,))


def _lane_bcast(v, lane):
    idx = jnp.full((L,), lane, jnp.int32)
    return lax.gather(v, idx[:, None], dimension_numbers=_GATHER_DNUMS,
                      slice_sizes=(1,),
                      mode=lax.GatherScatterMode.PROMISE_IN_BOUNDS)


def _rank_body(x_ref, xt_ref, o_ref):
    jlt = (lax.broadcasted_iota(jnp.int32, (IN_SIZE, IN_SIZE), 0)
           < lax.broadcasted_iota(jnp.int32, (IN_SIZE, IN_SIZE), 1))
    for r in range(RB):
        xi = x_ref[pl.ds(r, 1), :]
        xj = xt_ref[0, :, pl.ds(r, 1)]
        lt = xj < xi
        eq = xj == xi
        cnt = jnp.where(lt | (eq & jlt), 1.0, 0.0)
        rank = jnp.sum(cnt, axis=0)
        o_ref[pl.ds(r, 1), :] = rank[None, :].astype(jnp.int32)


def _ranks_tc(x, xt3):
    nb = x.shape[0]
    return pl.pallas_call(
        _rank_body,
        grid=(nb // RB,),
        in_specs=[
            pl.BlockSpec((RB, IN_SIZE), lambda g: (g, 0)),
            pl.BlockSpec((1, IN_SIZE, RB), lambda g: (g, 0, 0)),
        ],
        out_specs=pl.BlockSpec((RB, IN_SIZE), lambda g: (g, 0)),
        out_shape=jax.ShapeDtypeStruct((nb, IN_SIZE), jnp.int32),
    )(x, xt3)


def _make_snn_body(rpw):
  def _snn_body(x_hbm, w_hbm, rk_hbm, out_hbm,
                w_v, x_v, rk_v, sx_v, sxn_v, sidx_v, orow_v):
    wid = lax.axis_index("s") * 2 + lax.axis_index("c")
    pltpu.sync_copy(w_hbm, w_v)
    iota = lax.iota(jnp.int32, L)

    def row_body(r, _):
        row = wid * rpw + r
        pltpu.sync_copy(x_hbm.at[row], x_v)
        pltpu.sync_copy(rk_hbm.at[row], rk_v)

        sxn_v[pl.ds(IN_SIZE - L, L)] = jnp.full((L,), MAX_SPIKE_TIME,
                                                jnp.float32)

        def chunk_body(c, _):
            rank = rk_v[pl.ds(c * L, L)]
            xi = x_v[pl.ds(c * L, L)]
            plsc.store_scatter(sx_v, [rank], xi)
            plsc.store_scatter(sidx_v, [rank], iota + c * L)
            plsc.store_scatter(sxn_v, [rank - 1], xi, mask=rank >= 1)
            return 0

        lax.fori_loop(0, NCHUNK, chunk_body, 0)

        mn = [jnp.full((L,), MAX_SPIKE_TIME, jnp.float32)] * OCHUNK
        carryw = [jnp.zeros((L,), jnp.float32)] * OCHUNK
        carrywi = [jnp.zeros((L,), jnp.float32)] * OCHUNK
        for c in range(NBIG):
            cw_l = carryw
            cwi_l = carrywi

            def kchunk_body(kc, carry, _cw=cw_l, _cwi=cwi_l, _c=c):
                cw = list(carry[0:OCHUNK])
                cwi = list(carry[OCHUNK:2 * OCHUNK])
                mnl = list(carry[2 * OCHUNK:3 * OCHUNK])
                base = _c * CSZ + kc * L
                adr_c = sidx_v[pl.ds(base, L)]
                sx_c = sx_v[pl.ds(base, L)]
                sxn_c = sxn_v[pl.ds(base, L)]
                for l in range(L):
                    ridx = _lane_bcast(adr_c, l)
                    xk = _lane_bcast(sx_c, l)
                    xn = _lane_bcast(sxn_c, l)
                    for o in range(OCHUNK):
                        wv = plsc.load_gather(w_v, [ridx, iota + o * L])
                        cw[o] = cw[o] + wv
                        cwi[o] = cwi[o] + wv * xk
                        if _c == 0:
                            wsum = cw[o]
                            wisum = cwi[o]
                        else:
                            wsum = cw[o] + _cw[o]
                            wisum = cwi[o] + _cwi[o]
                        den = jnp.maximum(wsum - 1.0, 1e-10)
                        t = wisum / den
                        t = jnp.where(wsum < 1.0, MAX_SPIKE_TIME, t)
                        t = jnp.where(t < xk, MAX_SPIKE_TIME, t)
                        t = jnp.where(t > xn, MAX_SPIKE_TIME, t)
                        mnl[o] = jnp.minimum(mnl[o], t)
                return tuple(cw) + tuple(cwi) + tuple(mnl)

            zero = jnp.zeros((L,), jnp.float32)
            init = (zero,) * (2 * OCHUNK) + tuple(mn)
            res = lax.fori_loop(0, KC_PER_BIG, kchunk_body, init)
            mn = list(res[2 * OCHUNK:3 * OCHUNK])
            if c == 0:
                carryw = [res[o] for o in range(OCHUNK)]
                carrywi = [res[OCHUNK + o] for o in range(OCHUNK)]
            elif c < NBIG - 1:
                carryw = [carryw[o] + res[o] for o in range(OCHUNK)]
                carrywi = [carrywi[o] + res[OCHUNK + o] for o in range(OCHUNK)]

        for o in range(OCHUNK):
            orow_v[pl.ds(o * L, L)] = mn[o]
        pltpu.sync_copy(orow_v, out_hbm.at[row])
        return 0

    lax.fori_loop(0, rpw, row_body, 0)
  return _snn_body


def _snn_sc(x, w, ranks):
    nb = x.shape[0]
    mesh = plsc.VectorSubcoreMesh(core_axis_name="c", subcore_axis_name="s")
    f = pl.kernel(
        _make_snn_body(nb // NW),
        out_type=jax.ShapeDtypeStruct((nb, OUT_SIZE), jnp.float32),
        mesh=mesh,
        scratch_types=[
            pltpu.VMEM((IN_SIZE, OUT_SIZE), jnp.float32),
            pltpu.VMEM((IN_SIZE,), jnp.float32),
            pltpu.VMEM((IN_SIZE,), jnp.int32),
            pltpu.VMEM((IN_SIZE,), jnp.float32),
            pltpu.VMEM((IN_SIZE,), jnp.float32),
            pltpu.VMEM((IN_SIZE,), jnp.int32),
            pltpu.VMEM((OUT_SIZE,), jnp.float32),
        ],
        compiler_params=pltpu.CompilerParams(needs_layout_passes=False),
    )
    return f(x, w, ranks)


@jax.jit
def kernel(layer_in, weight):
    H = B // 2

    def ranks_half(xh):
        xt3 = xh.T.reshape(IN_SIZE, H // RB, RB).transpose(1, 0, 2)
        return _ranks_tc(xh, xt3)

    x1, x2 = layer_in[:H], layer_in[H:]
    o1 = _snn_sc(x1, weight, ranks_half(x1))
    o2 = _snn_sc(x2, weight, ranks_half(x2))
    return jnp.concatenate([o1, o2], axis=0)
